# baseline (speedup 1.0000x reference)
import functools

import jax
import jax.numpy as jnp
from jax import lax
from jax.experimental import pallas as pl
from jax.experimental.pallas import tpu as pltpu
from jax.experimental.pallas import tpu_sc as plsc

VOCAB = 100000
EMBED = 256
HIDDEN = 1024
FC1 = 128
OUT = 2
B = 1024
L = 20
LANE = 128


@functools.cache
def _make_sc_gather(V, D, N):
    info = plsc.get_sparse_core_info()
    nw = info.num_cores * info.num_subcores
    n_per_w = N // nw
    assert N % (8 * nw) == 0
    CH = 128
    assert n_per_w % CH == 0
    n_ch = n_per_w // CH
    mesh = plsc.VectorSubcoreMesh(core_axis_name="c", subcore_axis_name="s")

    @functools.partial(
        pl.kernel,
        mesh=mesh,
        out_type=jax.ShapeDtypeStruct((N, D), jnp.float32),
        scratch_types=[
            pltpu.VMEM((CH,), jnp.int32),
            pltpu.VMEM((CH, D), jnp.float32),
            pltpu.SemaphoreType.DMA,
        ],
    )
    def gather(table_hbm, idx_hbm, out_hbm, idx_v, rows_v, sem):
        wid = lax.axis_index("s") * info.num_cores + lax.axis_index("c")
        base = wid * n_per_w
        for c in range(n_ch):
            off = base + c * CH
            pltpu.sync_copy(idx_hbm.at[pl.ds(off, CH)], idx_v)
            pltpu.async_copy(table_hbm.at[idx_v], rows_v, sem).wait()
            pltpu.sync_copy(rows_v, out_hbm.at[pl.ds(off, CH)])

    return gather


def _rnn_body(emb_ref, wih_ref, whh_ref, bias_ref, fc1w_ref, fc1b_ref,
              fc2w_ref, fc2b_ref, out_ref, h_ref, xp_ref):
    t = pl.program_id(0)

    @pl.when(t > 0)
    def _():
        acc = xp_ref[...] + bias_ref[...]
        acc = acc + jnp.dot(h_ref[...], whh_ref[...],
                            preferred_element_type=jnp.float32)
        h_new = jnp.tanh(acc)
        h_ref[...] = h_new.astype(jnp.bfloat16)

        @pl.when(t == L)
        def _():
            feat = jnp.dot(h_new, fc1w_ref[...],
                           preferred_element_type=jnp.float32)
            feat = jnp.maximum(feat + fc1b_ref[...], 0.0)
            logits = jnp.dot(feat, fc2w_ref[...],
                             preferred_element_type=jnp.float32)
            logits = logits + fc2b_ref[...]
            m = jnp.max(logits, axis=1, keepdims=True)
            e = jnp.exp(logits - m)
            out_ref[...] = e / jnp.sum(e, axis=1, keepdims=True)

    @pl.when(t == 0)
    def _():
        h_ref[...] = jnp.zeros_like(h_ref)

    @pl.when(t < L)
    def _():
        xp_ref[...] = jnp.dot(emb_ref[...], wih_ref[...],
                              preferred_element_type=jnp.float32)


@functools.partial(jax.jit, static_argnums=())
def _rnn_mlp(emb, wih_t, whh_t, bias, fc1w_t, fc1b, fc2w_pad, fc2b_pad):
    return pl.pallas_call(
        _rnn_body,
        grid=(L + 1,),
        in_specs=[
            pl.BlockSpec((B, EMBED), lambda t: (jnp.minimum(t, L - 1), 0)),
            pl.BlockSpec((EMBED, HIDDEN), lambda t: (0, 0)),
            pl.BlockSpec((HIDDEN, HIDDEN), lambda t: (0, 0)),
            pl.BlockSpec((1, HIDDEN), lambda t: (0, 0)),
            pl.BlockSpec((HIDDEN, FC1), lambda t: (0, 0)),
            pl.BlockSpec((1, FC1), lambda t: (0, 0)),
            pl.BlockSpec((FC1, LANE), lambda t: (0, 0)),
            pl.BlockSpec((1, LANE), lambda t: (0, 0)),
        ],
        out_specs=pl.BlockSpec((B, LANE), lambda t: (0, 0)),
        out_shape=jax.ShapeDtypeStruct((B, LANE), jnp.float32),
        scratch_shapes=[pltpu.VMEM((B, HIDDEN), jnp.bfloat16),
                        pltpu.VMEM((B, HIDDEN), jnp.float32)],
        compiler_params=pltpu.CompilerParams(
            dimension_semantics=("arbitrary",)),
    )(emb, wih_t, whh_t, bias, fc1w_t, fc1b, fc2w_pad, fc2b_pad)


def kernel(x, embed_table, W_ih, b_ih, W_hh, b_hh, fc1_W, fc1_b, fc2_W, fc2_b):
    idx = jnp.swapaxes(x, 0, 1).reshape(-1).astype(jnp.int32)
    emb = _make_sc_gather(VOCAB, EMBED, L * B)(embed_table, idx)

    bias = (b_ih + b_hh).reshape(1, HIDDEN)
    fc2w_pad = jnp.zeros((FC1, LANE), jnp.float32).at[:, :OUT].set(fc2_W.T)
    fc2b_pad = jnp.full((1, LANE), -1e30, jnp.float32).at[0, :OUT].set(fc2_b)
    probs = _rnn_mlp(emb, W_ih.T, W_hh.T.astype(jnp.bfloat16), bias, fc1_W.T,
                     fc1_b.reshape(1, FC1), fc2w_pad, fc2b_pad)
    return probs[:, :OUT]


# speedup vs baseline: 1.0297x; 1.0297x over previous
import functools

import jax
import jax.numpy as jnp
from jax import lax
from jax.experimental import pallas as pl
from jax.experimental.pallas import tpu as pltpu
from jax.experimental.pallas import tpu_sc as plsc

VOCAB = 100000
EMBED = 256
HIDDEN = 1024
FC1 = 128
OUT = 2
B = 1024
L = 20
LANE = 128


@functools.cache
def _make_sc_gather(V, D, N):
    info = plsc.get_sparse_core_info()
    nw = info.num_cores * info.num_subcores
    n_per_w = N // nw
    assert N % (8 * nw) == 0
    CH = 128
    assert n_per_w % CH == 0
    n_ch = n_per_w // CH
    mesh = plsc.VectorSubcoreMesh(core_axis_name="c", subcore_axis_name="s")

    @functools.partial(
        pl.kernel,
        mesh=mesh,
        out_type=jax.ShapeDtypeStruct((N, D), jnp.float32),
        scratch_types=[
            pltpu.VMEM((CH,), jnp.int32),
            pltpu.VMEM((CH, D), jnp.float32),
            pltpu.SemaphoreType.DMA,
        ],
    )
    def gather(table_hbm, idx_hbm, out_hbm, idx_v, rows_v, sem):
        wid = lax.axis_index("s") * info.num_cores + lax.axis_index("c")
        base = wid * n_per_w
        for c in range(n_ch):
            off = base + c * CH
            pltpu.sync_copy(idx_hbm.at[pl.ds(off, CH)], idx_v)
            pltpu.async_copy(table_hbm.at[idx_v], rows_v, sem).wait()
            pltpu.sync_copy(rows_v, out_hbm.at[pl.ds(off, CH)])

    return gather


def _rnn_body(emb_ref, wih_ref, whh_ref, bias_ref, fc1w_ref, fc1b_ref,
              fc2w_ref, fc2b_ref, out_ref, h_ref, xp_ref):
    t = pl.program_id(0)

    acc = xp_ref[...] + bias_ref[...]
    acc = acc + jnp.dot(h_ref[...], whh_ref[...],
                        preferred_element_type=jnp.float32)
    h_new = jnp.tanh(acc)
    h_ref[...] = jnp.where(t > 0, h_new, 0.0).astype(jnp.bfloat16)
    xp_ref[...] = jnp.dot(emb_ref[...], wih_ref[...],
                          preferred_element_type=jnp.float32)

    @pl.when(t == L)
    def _():
        feat = jnp.dot(h_new, fc1w_ref[...],
                       preferred_element_type=jnp.float32)
        feat = jnp.maximum(feat + fc1b_ref[...], 0.0)
        logits = jnp.dot(feat, fc2w_ref[...],
                         preferred_element_type=jnp.float32)
        logits = logits + fc2b_ref[...]
        m = jnp.max(logits, axis=1, keepdims=True)
        e = jnp.exp(logits - m)
        out_ref[...] = e / jnp.sum(e, axis=1, keepdims=True)


@functools.partial(jax.jit, static_argnums=())
def _rnn_mlp(emb, wih_t, whh_t, bias, fc1w_t, fc1b, fc2w_pad, fc2b_pad):
    return pl.pallas_call(
        _rnn_body,
        grid=(L + 1,),
        in_specs=[
            pl.BlockSpec((B, EMBED), lambda t: (jnp.minimum(t, L - 1), 0)),
            pl.BlockSpec((EMBED, HIDDEN), lambda t: (0, 0)),
            pl.BlockSpec((HIDDEN, HIDDEN), lambda t: (0, 0)),
            pl.BlockSpec((1, HIDDEN), lambda t: (0, 0)),
            pl.BlockSpec((HIDDEN, FC1), lambda t: (0, 0)),
            pl.BlockSpec((1, FC1), lambda t: (0, 0)),
            pl.BlockSpec((FC1, LANE), lambda t: (0, 0)),
            pl.BlockSpec((1, LANE), lambda t: (0, 0)),
        ],
        out_specs=pl.BlockSpec((B, LANE), lambda t: (0, 0)),
        out_shape=jax.ShapeDtypeStruct((B, LANE), jnp.float32),
        scratch_shapes=[pltpu.VMEM((B, HIDDEN), jnp.bfloat16),
                        pltpu.VMEM((B, HIDDEN), jnp.float32)],
        compiler_params=pltpu.CompilerParams(
            dimension_semantics=("arbitrary",)),
    )(emb, wih_t, whh_t, bias, fc1w_t, fc1b, fc2w_pad, fc2b_pad)


def kernel(x, embed_table, W_ih, b_ih, W_hh, b_hh, fc1_W, fc1_b, fc2_W, fc2_b):
    idx = jnp.swapaxes(x, 0, 1).reshape(-1).astype(jnp.int32)
    emb = _make_sc_gather(VOCAB, EMBED, L * B)(embed_table, idx)

    bias = (b_ih + b_hh).reshape(1, HIDDEN)
    fc2w_pad = jnp.zeros((FC1, LANE), jnp.float32).at[:, :OUT].set(fc2_W.T)
    fc2b_pad = jnp.full((1, LANE), -1e30, jnp.float32).at[0, :OUT].set(fc2_b)
    probs = _rnn_mlp(emb, W_ih.T, W_hh.T.astype(jnp.bfloat16), bias, fc1_W.T,
                     fc1_b.reshape(1, FC1), fc2w_pad, fc2b_pad)
    return probs[:, :OUT]


# speedup vs baseline: 1.0355x; 1.0056x over previous
import functools

import jax
import jax.numpy as jnp
from jax import lax
from jax.experimental import pallas as pl
from jax.experimental.pallas import tpu as pltpu
from jax.experimental.pallas import tpu_sc as plsc

VOCAB = 100000
EMBED = 256
HIDDEN = 1024
FC1 = 128
OUT = 2
B = 1024
L = 20
LANE = 128


@functools.cache
def _make_sc_gather(V, D, N):
    info = plsc.get_sparse_core_info()
    nw = info.num_cores * info.num_subcores
    n_per_w = N // nw
    assert N % (8 * nw) == 0
    CH = 128
    assert n_per_w % CH == 0
    n_ch = n_per_w // CH
    mesh = plsc.VectorSubcoreMesh(core_axis_name="c", subcore_axis_name="s")

    @functools.partial(
        pl.kernel,
        mesh=mesh,
        out_type=jax.ShapeDtypeStruct((N, D), jnp.float32),
        scratch_types=[
            pltpu.VMEM((CH,), jnp.int32),
            pltpu.VMEM((CH, D), jnp.float32),
            pltpu.SemaphoreType.DMA,
        ],
    )
    def gather(table_hbm, idx_hbm, out_hbm, idx_v, rows_v, sem):
        wid = lax.axis_index("s") * info.num_cores + lax.axis_index("c")
        base = wid * n_per_w
        for c in range(n_ch):
            off = base + c * CH
            pltpu.sync_copy(idx_hbm.at[pl.ds(off, CH)], idx_v)
            pltpu.async_copy(table_hbm.at[idx_v], rows_v, sem).wait()
            pltpu.sync_copy(rows_v, out_hbm.at[pl.ds(off, CH)])

    return gather


def _rnn_body(emb_ref, wih_ref, whh_ref, bias_ref, fc1w_ref, fc1b_ref,
              fc2w_ref, fc2b_ref, out_ref, h_ref, xp_ref):
    t = pl.program_id(0)

    acc = xp_ref[...] + bias_ref[...]
    acc = acc + jnp.dot(h_ref[...], whh_ref[...],
                        preferred_element_type=jnp.float32)
    h_new = jnp.tanh(acc)
    h_ref[...] = jnp.where(t > 0, h_new, 0.0).astype(jnp.bfloat16)
    xp_ref[...] = jnp.dot(emb_ref[...].astype(jnp.bfloat16), wih_ref[...],
                          preferred_element_type=jnp.float32)

    @pl.when(t == L)
    def _():
        feat = jnp.dot(h_new, fc1w_ref[...],
                       preferred_element_type=jnp.float32)
        feat = jnp.maximum(feat + fc1b_ref[...], 0.0)
        logits = jnp.dot(feat, fc2w_ref[...],
                         preferred_element_type=jnp.float32)
        logits = logits + fc2b_ref[...]
        m = jnp.max(logits, axis=1, keepdims=True)
        e = jnp.exp(logits - m)
        out_ref[...] = e / jnp.sum(e, axis=1, keepdims=True)


@functools.partial(jax.jit, static_argnums=())
def _rnn_mlp(emb, wih_t, whh_t, bias, fc1w_t, fc1b, fc2w_pad, fc2b_pad):
    return pl.pallas_call(
        _rnn_body,
        grid=(L + 1,),
        in_specs=[
            pl.BlockSpec((B, EMBED), lambda t: (jnp.minimum(t, L - 1), 0)),
            pl.BlockSpec((EMBED, HIDDEN), lambda t: (0, 0)),
            pl.BlockSpec((HIDDEN, HIDDEN), lambda t: (0, 0)),
            pl.BlockSpec((1, HIDDEN), lambda t: (0, 0)),
            pl.BlockSpec((HIDDEN, FC1), lambda t: (0, 0)),
            pl.BlockSpec((1, FC1), lambda t: (0, 0)),
            pl.BlockSpec((FC1, LANE), lambda t: (0, 0)),
            pl.BlockSpec((1, LANE), lambda t: (0, 0)),
        ],
        out_specs=pl.BlockSpec((B, LANE), lambda t: (0, 0)),
        out_shape=jax.ShapeDtypeStruct((B, LANE), jnp.float32),
        scratch_shapes=[pltpu.VMEM((B, HIDDEN), jnp.bfloat16),
                        pltpu.VMEM((B, HIDDEN), jnp.float32)],
        compiler_params=pltpu.CompilerParams(
            dimension_semantics=("arbitrary",)),
    )(emb, wih_t, whh_t, bias, fc1w_t, fc1b, fc2w_pad, fc2b_pad)


def kernel(x, embed_table, W_ih, b_ih, W_hh, b_hh, fc1_W, fc1_b, fc2_W, fc2_b):
    idx = jnp.swapaxes(x, 0, 1).reshape(-1).astype(jnp.int32)
    emb = _make_sc_gather(VOCAB, EMBED, L * B)(embed_table, idx)

    bias = (b_ih + b_hh).reshape(1, HIDDEN)
    fc2w_pad = jnp.zeros((FC1, LANE), jnp.float32).at[:, :OUT].set(fc2_W.T)
    fc2b_pad = jnp.full((1, LANE), -1e30, jnp.float32).at[0, :OUT].set(fc2_b)
    probs = _rnn_mlp(emb, W_ih.T.astype(jnp.bfloat16),
                     W_hh.T.astype(jnp.bfloat16), bias, fc1_W.T,
                     fc1_b.reshape(1, FC1), fc2w_pad, fc2b_pad)
    return probs[:, :OUT]


# speedup vs baseline: 1.1341x; 1.0952x over previous
import functools

import jax
import jax.numpy as jnp
from jax import lax
from jax.experimental import pallas as pl
from jax.experimental.pallas import tpu as pltpu
from jax.experimental.pallas import tpu_sc as plsc

VOCAB = 100000
EMBED = 256
HIDDEN = 1024
FC1 = 128
OUT = 2
B = 1024
L = 20
LANE = 128
SPLIT = 6


@functools.cache
def _make_sc_gather(V, D, N):
    info = plsc.get_sparse_core_info()
    nw = info.num_cores * info.num_subcores
    n_per_w = N // nw
    assert N % (8 * nw) == 0
    ch = 8
    for c in range(8, 129, 8):
        if n_per_w % c == 0:
            ch = c
    n_ch = n_per_w // ch
    mesh = plsc.VectorSubcoreMesh(core_axis_name="c", subcore_axis_name="s")

    @functools.partial(
        pl.kernel,
        mesh=mesh,
        out_type=jax.ShapeDtypeStruct((N, D), jnp.float32),
        scratch_types=[
            pltpu.VMEM((ch,), jnp.int32),
            pltpu.VMEM((ch, D), jnp.float32),
            pltpu.SemaphoreType.DMA,
        ],
    )
    def gather(table_hbm, idx_hbm, out_hbm, idx_v, rows_v, sem):
        wid = lax.axis_index("s") * info.num_cores + lax.axis_index("c")
        base = wid * n_per_w
        for c in range(n_ch):
            off = base + c * ch
            pltpu.sync_copy(idx_hbm.at[pl.ds(off, ch)], idx_v)
            pltpu.async_copy(table_hbm.at[idx_v], rows_v, sem).wait()
            pltpu.sync_copy(rows_v, out_hbm.at[pl.ds(off, ch)])

    return gather


def _rnn_seg1_body(emb_ref, wih_ref, whh_ref, bias_ref, out_ref, h_ref):
    t = pl.program_id(0)

    @pl.when(t == 0)
    def _():
        h_ref[...] = jnp.zeros_like(h_ref)

    acc = jnp.dot(emb_ref[...].astype(jnp.bfloat16), wih_ref[...],
                  preferred_element_type=jnp.float32)
    acc = acc + jnp.dot(h_ref[...], whh_ref[...],
                        preferred_element_type=jnp.float32)
    h_new = jnp.tanh(acc + bias_ref[...]).astype(jnp.bfloat16)
    h_ref[...] = h_new

    @pl.when(t == SPLIT - 1)
    def _():
        out_ref[...] = h_new


def _rnn_seg2_body(emb_ref, h0_ref, wih_ref, whh_ref, bias_ref, fc1w_ref,
                   fc1b_ref, fc2w_ref, fc2b_ref, out_ref, h_ref):
    t = pl.program_id(0)
    T = L - SPLIT

    @pl.when(t == 0)
    def _():
        h_ref[...] = h0_ref[...]

    acc = jnp.dot(emb_ref[...].astype(jnp.bfloat16), wih_ref[...],
                  preferred_element_type=jnp.float32)
    acc = acc + jnp.dot(h_ref[...], whh_ref[...],
                        preferred_element_type=jnp.float32)
    h_new = jnp.tanh(acc + bias_ref[...])
    h_ref[...] = h_new.astype(jnp.bfloat16)

    @pl.when(t == T - 1)
    def _():
        feat = jnp.dot(h_new, fc1w_ref[...],
                       preferred_element_type=jnp.float32)
        feat = jnp.maximum(feat + fc1b_ref[...], 0.0)
        logits = jnp.dot(feat, fc2w_ref[...],
                         preferred_element_type=jnp.float32)
        logits = logits + fc2b_ref[...]
        m = jnp.max(logits, axis=1, keepdims=True)
        e = jnp.exp(logits - m)
        out_ref[...] = e / jnp.sum(e, axis=1, keepdims=True)


_FULL = lambda t: (0, 0)


def _rnn_seg1(emb1, wih_t, whh_t, bias):
    return pl.pallas_call(
        _rnn_seg1_body,
        grid=(SPLIT,),
        in_specs=[
            pl.BlockSpec((B, EMBED), lambda t: (t, 0)),
            pl.BlockSpec((EMBED, HIDDEN), _FULL),
            pl.BlockSpec((HIDDEN, HIDDEN), _FULL),
            pl.BlockSpec((1, HIDDEN), _FULL),
        ],
        out_specs=pl.BlockSpec((B, HIDDEN), _FULL),
        out_shape=jax.ShapeDtypeStruct((B, HIDDEN), jnp.bfloat16),
        scratch_shapes=[pltpu.VMEM((B, HIDDEN), jnp.bfloat16)],
        compiler_params=pltpu.CompilerParams(
            dimension_semantics=("arbitrary",)),
    )(emb1, wih_t, whh_t, bias)


def _rnn_seg2(emb2, h0, wih_t, whh_t, bias, fc1w_t, fc1b, fc2w_pad, fc2b_pad):
    return pl.pallas_call(
        _rnn_seg2_body,
        grid=(L - SPLIT,),
        in_specs=[
            pl.BlockSpec((B, EMBED), lambda t: (t, 0)),
            pl.BlockSpec((B, HIDDEN), _FULL),
            pl.BlockSpec((EMBED, HIDDEN), _FULL),
            pl.BlockSpec((HIDDEN, HIDDEN), _FULL),
            pl.BlockSpec((1, HIDDEN), _FULL),
            pl.BlockSpec((HIDDEN, FC1), _FULL),
            pl.BlockSpec((1, FC1), _FULL),
            pl.BlockSpec((FC1, LANE), _FULL),
            pl.BlockSpec((1, LANE), _FULL),
        ],
        out_specs=pl.BlockSpec((B, LANE), _FULL),
        out_shape=jax.ShapeDtypeStruct((B, LANE), jnp.float32),
        scratch_shapes=[pltpu.VMEM((B, HIDDEN), jnp.bfloat16)],
        compiler_params=pltpu.CompilerParams(
            dimension_semantics=("arbitrary",)),
    )(emb2, h0, wih_t, whh_t, bias, fc1w_t, fc1b, fc2w_pad, fc2b_pad)


def kernel(x, embed_table, W_ih, b_ih, W_hh, b_hh, fc1_W, fc1_b, fc2_W, fc2_b):
    idx = jnp.swapaxes(x, 0, 1).reshape(-1).astype(jnp.int32)
    idx1 = idx[: SPLIT * B]
    idx2 = idx[SPLIT * B:]
    emb1 = _make_sc_gather(VOCAB, EMBED, SPLIT * B)(embed_table, idx1)
    emb2 = _make_sc_gather(VOCAB, EMBED, (L - SPLIT) * B)(embed_table, idx2)

    wih_t = W_ih.T.astype(jnp.bfloat16)
    whh_t = W_hh.T.astype(jnp.bfloat16)
    bias = (b_ih + b_hh).reshape(1, HIDDEN)
    fc2w_pad = jnp.zeros((FC1, LANE), jnp.float32).at[:, :OUT].set(fc2_W.T)
    fc2b_pad = jnp.full((1, LANE), -1e30, jnp.float32).at[0, :OUT].set(fc2_b)

    h_mid = _rnn_seg1(emb1, wih_t, whh_t, bias)
    probs = _rnn_seg2(emb2, h_mid, wih_t, whh_t, bias, fc1_W.T,
                      fc1_b.reshape(1, FC1), fc2w_pad, fc2b_pad)
    return probs[:, :OUT]


# speedup vs baseline: 1.1456x; 1.0101x over previous
import functools

import jax
import jax.numpy as jnp
from jax import lax
from jax.experimental import pallas as pl
from jax.experimental.pallas import tpu as pltpu
from jax.experimental.pallas import tpu_sc as plsc

VOCAB = 100000
EMBED = 256
HIDDEN = 1024
FC1 = 128
OUT = 2
B = 1024
L = 20
LANE = 128
SPLIT = 6


@functools.cache
def _make_sc_gather(V, D, N):
    info = plsc.get_sparse_core_info()
    nw = info.num_cores * info.num_subcores
    n_per_w = N // nw
    assert N % (8 * nw) == 0
    ch = 8
    for c in range(8, 129, 8):
        if n_per_w % c == 0:
            ch = c
    n_ch = n_per_w // ch
    mesh = plsc.VectorSubcoreMesh(core_axis_name="c", subcore_axis_name="s")

    @functools.partial(
        pl.kernel,
        mesh=mesh,
        out_type=jax.ShapeDtypeStruct((N, D), jnp.float32),
        scratch_types=[
            pltpu.VMEM((ch,), jnp.int32),
            pltpu.VMEM((ch, D), jnp.float32),
            pltpu.SemaphoreType.DMA,
        ],
    )
    def gather(table_hbm, idx_hbm, out_hbm, idx_v, rows_v, sem):
        wid = lax.axis_index("s") * info.num_cores + lax.axis_index("c")
        base = wid * n_per_w
        for c in range(n_ch):
            off = base + c * ch
            pltpu.sync_copy(idx_hbm.at[pl.ds(off, ch)], idx_v)
            pltpu.async_copy(table_hbm.at[idx_v], rows_v, sem).wait()
            pltpu.sync_copy(rows_v, out_hbm.at[pl.ds(off, ch)])

    return gather


_NT = (((1,), (1,)), ((), ()))


def _rnn_seg1_body(emb_ref, wih_ref, whh_ref, bias_ref, out_ref, h_ref):
    t = pl.program_id(0)

    @pl.when(t == 0)
    def _():
        h_ref[...] = jnp.zeros_like(h_ref)

    acc = lax.dot_general(emb_ref[...].astype(jnp.bfloat16), wih_ref[...],
                          _NT, preferred_element_type=jnp.float32)
    acc = acc + lax.dot_general(h_ref[...], whh_ref[...], _NT,
                                preferred_element_type=jnp.float32)
    h_new = jnp.tanh(acc + bias_ref[...]).astype(jnp.bfloat16)
    h_ref[...] = h_new

    @pl.when(t == SPLIT - 1)
    def _():
        out_ref[...] = h_new


def _rnn_seg2_body(emb_ref, h0_ref, wih_ref, whh_ref, bias_ref, fc1w_ref,
                   fc1b_ref, fc2w_ref, fc2b_ref, out_ref, h_ref):
    t = pl.program_id(0)
    T = L - SPLIT

    @pl.when(t == 0)
    def _():
        h_ref[...] = h0_ref[...]

    acc = lax.dot_general(emb_ref[...].astype(jnp.bfloat16), wih_ref[...],
                          _NT, preferred_element_type=jnp.float32)
    acc = acc + lax.dot_general(h_ref[...], whh_ref[...], _NT,
                                preferred_element_type=jnp.float32)
    h_new = jnp.tanh(acc + bias_ref[...])
    h_ref[...] = h_new.astype(jnp.bfloat16)

    @pl.when(t == T - 1)
    def _():
        feat = lax.dot_general(h_new, fc1w_ref[...], _NT,
                               preferred_element_type=jnp.float32)
        feat = jnp.maximum(feat + fc1b_ref[...], 0.0)
        logits = lax.dot_general(feat, fc2w_ref[...], _NT,
                                 preferred_element_type=jnp.float32)
        logits = logits + fc2b_ref[...]
        m = jnp.max(logits, axis=1, keepdims=True)
        e = jnp.exp(logits - m)
        out_ref[...] = e / jnp.sum(e, axis=1, keepdims=True)


_FULL = lambda t: (0, 0)


def _rnn_seg1(emb1, wih_t, whh_t, bias):
    return pl.pallas_call(
        _rnn_seg1_body,
        grid=(SPLIT,),
        in_specs=[
            pl.BlockSpec((B, EMBED), lambda t: (t, 0)),
            pl.BlockSpec((HIDDEN, EMBED), _FULL),
            pl.BlockSpec((HIDDEN, HIDDEN), _FULL),
            pl.BlockSpec((1, HIDDEN), _FULL),
        ],
        out_specs=pl.BlockSpec((B, HIDDEN), _FULL),
        out_shape=jax.ShapeDtypeStruct((B, HIDDEN), jnp.bfloat16),
        scratch_shapes=[pltpu.VMEM((B, HIDDEN), jnp.bfloat16)],
        compiler_params=pltpu.CompilerParams(
            dimension_semantics=("arbitrary",)),
    )(emb1, wih_t, whh_t, bias)


def _rnn_seg2(emb2, h0, wih_t, whh_t, bias, fc1w_t, fc1b, fc2w_pad, fc2b_pad):
    return pl.pallas_call(
        _rnn_seg2_body,
        grid=(L - SPLIT,),
        in_specs=[
            pl.BlockSpec((B, EMBED), lambda t: (t, 0)),
            pl.BlockSpec((B, HIDDEN), _FULL),
            pl.BlockSpec((HIDDEN, EMBED), _FULL),
            pl.BlockSpec((HIDDEN, HIDDEN), _FULL),
            pl.BlockSpec((1, HIDDEN), _FULL),
            pl.BlockSpec((FC1, HIDDEN), _FULL),
            pl.BlockSpec((1, FC1), _FULL),
            pl.BlockSpec((LANE, FC1), _FULL),
            pl.BlockSpec((1, LANE), _FULL),
        ],
        out_specs=pl.BlockSpec((B, LANE), _FULL),
        out_shape=jax.ShapeDtypeStruct((B, LANE), jnp.float32),
        scratch_shapes=[pltpu.VMEM((B, HIDDEN), jnp.bfloat16)],
        compiler_params=pltpu.CompilerParams(
            dimension_semantics=("arbitrary",)),
    )(emb2, h0, wih_t, whh_t, bias, fc1w_t, fc1b, fc2w_pad, fc2b_pad)


def kernel(x, embed_table, W_ih, b_ih, W_hh, b_hh, fc1_W, fc1_b, fc2_W, fc2_b):
    idx = jnp.swapaxes(x, 0, 1).reshape(-1).astype(jnp.int32)
    idx1 = idx[: SPLIT * B]
    idx2 = idx[SPLIT * B:]
    emb1 = _make_sc_gather(VOCAB, EMBED, SPLIT * B)(embed_table, idx1)
    emb2 = _make_sc_gather(VOCAB, EMBED, (L - SPLIT) * B)(embed_table, idx2)

    wih = W_ih.astype(jnp.bfloat16)
    whh = W_hh.astype(jnp.bfloat16)
    bias = (b_ih + b_hh).reshape(1, HIDDEN)
    fc2w_pad = jnp.pad(fc2_W, ((0, LANE - OUT), (0, 0)))
    fc2b_pad = jnp.pad(fc2_b, (0, LANE - OUT),
                       constant_values=-1e30).reshape(1, LANE)

    h_mid = _rnn_seg1(emb1, wih, whh, bias)
    probs = _rnn_seg2(emb2, h_mid, wih, whh, bias, fc1_W,
                      fc1_b.reshape(1, FC1), fc2w_pad, fc2b_pad)
    return probs[:, :OUT]
